# baseline (speedup 1.0000x reference)
import jax
import jax.numpy as jnp
from jax.experimental import pallas as pl

N, L, H = 256, 2, 512

_DN = (((1,), (1,)), ((), ()))


def _gru_cell(x, h, wih_ref, whh_ref, bih, bhh):
    gi = jax.lax.dot_general(x, wih_ref[...], _DN,
                             preferred_element_type=jnp.float32) + bih
    gh = jax.lax.dot_general(h, whh_ref[...], _DN,
                             preferred_element_type=jnp.float32) + bhh
    i_r = gi[:, 0 * H:1 * H]
    i_z = gi[:, 1 * H:2 * H]
    i_n = gi[:, 2 * H:3 * H]
    h_r = gh[:, 0 * H:1 * H]
    h_z = gh[:, 1 * H:2 * H]
    h_n = gh[:, 2 * H:3 * H]
    r = jax.nn.sigmoid(i_r + h_r)
    z = jax.nn.sigmoid(i_z + h_z)
    n = jnp.tanh(i_n + r * h_n)
    return (1.0 - z) * n + z * h


def _gru2_kernel(x_ref, h_ref, m_ref,
                 wih0_ref, whh0_ref, bih0_ref, bhh0_ref,
                 wih1_ref, whh1_ref, bih1_ref, bhh1_ref,
                 out_ref, newh_ref):
    x = x_ref[...]
    m = m_ref[...]
    h0 = h_ref[:, 0, :] * m
    h1 = h_ref[:, 1, :] * m
    h0n = _gru_cell(x, h0, wih0_ref, whh0_ref, bih0_ref[...], bhh0_ref[...])
    h1n = _gru_cell(h0n, h1, wih1_ref, whh1_ref, bih1_ref[...], bhh1_ref[...])
    out_ref[...] = h1n
    newh_ref[:, 0, :] = h0n
    newh_ref[:, 1, :] = h1n


def kernel(x, hidden_states, masks, W_ih0, W_hh0, b_ih0, b_hh0,
           W_ih1, W_hh1, b_ih1, b_hh1):
    m = masks.astype(jnp.float32)
    out, new_h = pl.pallas_call(
        _gru2_kernel,
        out_shape=(
            jax.ShapeDtypeStruct((N, H), jnp.float32),
            jax.ShapeDtypeStruct((N, L, H), jnp.float32),
        ),
    )(x, hidden_states, m,
      W_ih0, W_hh0, b_ih0.reshape(1, 3 * H), b_hh0.reshape(1, 3 * H),
      W_ih1, W_hh1, b_ih1.reshape(1, 3 * H), b_hh1.reshape(1, 3 * H))
    return (out, new_h)
